# baseline (speedup 1.0000x reference)
import functools

import jax
import jax.numpy as jnp
from jax import lax
from jax.experimental import pallas as pl
from jax.experimental.pallas import tpu as pltpu
from jax.experimental.pallas import tpu_sc as plsc

VOCAB = 100000
DIM = 128
BATCH = 16384

NUM_CORES = 2
NUM_SUBCORES = 16
NW = NUM_CORES * NUM_SUBCORES
BPW = BATCH // NW
CH = 128
NCH = BPW // CH
NBUF = 2
LANES = 16


PSTRIDE = LANES + 1
PGROUP = PSTRIDE * LANES


def _lane_permute(x, idx):
    dnums = lax.GatherDimensionNumbers(
        offset_dims=(), collapsed_slice_dims=(0,), start_index_map=(0,))
    return lax.gather(
        x, idx[:, None], dnums, slice_sizes=(1,),
        mode=lax.GatherScatterMode.PROMISE_IN_BOUNDS)


def _dot_group(a_v, b_v, slot, row0):
    lanes = lax.broadcasted_iota(jnp.int32, (LANES,), 0)
    perms = [lanes ^ step for step in (8, 4, 2, 1)]

    @plsc.parallel_loop(0, LANES, unroll=8, carry=jnp.zeros((LANES,), jnp.float32))
    def res(r, out):
        row = row0 + r
        acc = None
        for k in range(DIM // LANES):
            av = a_v[slot, row, pl.ds(k * LANES, LANES)]
            bv = b_v[slot, row, pl.ds(k * LANES, LANES)]
            prod = av * bv
            acc = prod if acc is None else acc + prod
        for perm in perms:
            acc = acc + _lane_permute(acc, perm)
        return jnp.where(lanes == r, acc, out)

    return res


def _dot_group_cols(a_v, b_v, row0):
    rows = lax.broadcasted_iota(jnp.int32, (LANES,), 0) + row0
    zeros = jnp.zeros((LANES,), jnp.float32)
    nacc = 8

    def body(i, accs):
        d0 = i * nacc
        col = jnp.full((LANES,), d0, dtype=jnp.int32)
        outs = []
        for j, acc in enumerate(accs):
            cj = col + j
            av = plsc.load_gather(a_v, [rows, cj])
            bv = plsc.load_gather(b_v, [rows, cj])
            outs.append(acc + av * bv)
        return tuple(outs)

    accs = lax.fori_loop(0, DIM // nacc, body, (zeros,) * nacc)
    s01 = (accs[0] + accs[1]) + (accs[2] + accs[3])
    s23 = (accs[4] + accs[5]) + (accs[6] + accs[7])
    return s01 + s23


@functools.partial(
    pl.kernel,
    mesh=plsc.VectorSubcoreMesh(core_axis_name="c", subcore_axis_name="s"),
    out_type=jax.ShapeDtypeStruct((BATCH,), jnp.float32),
    scratch_types=[
        pltpu.VMEM((BPW,), jnp.int32),
        pltpu.VMEM((BPW,), jnp.int32),
        pltpu.VMEM((NBUF, CH, DIM), jnp.float32),
        pltpu.VMEM((NBUF, CH, DIM), jnp.float32),
        pltpu.VMEM((BPW,), jnp.float32),
        pltpu.SemaphoreType.DMA,
        pltpu.SemaphoreType.DMA,
    ],
    compiler_params=pltpu.CompilerParams(needs_layout_passes=False),
)
def _w2v_kernel(center_hbm, context_hbm, w_in_hbm, w_out_hbm, out_hbm,
                cidx_v, xidx_v, a_v, b_v, out_v, sem0, sem1):
    wid = lax.axis_index("s") * NUM_CORES + lax.axis_index("c")
    base = wid * BPW

    sems = (sem0, sem1)

    hc = pltpu.async_copy(center_hbm.at[pl.ds(base, BPW)], cidx_v, sem0)
    hx = pltpu.async_copy(context_hbm.at[pl.ds(base, BPW)], xidx_v, sem1)
    hc.wait()
    hx.wait()

    def start(c):
        slot = c % NBUF
        ha = pltpu.async_copy(
            w_in_hbm.at[cidx_v.at[pl.ds(c * CH, CH)]], a_v.at[slot], sems[slot])
        hb = pltpu.async_copy(
            w_out_hbm.at[xidx_v.at[pl.ds(c * CH, CH)]], b_v.at[slot], sems[slot])
        return ha, hb

    pending = [start(c) for c in range(NBUF - 1)]
    for c in range(NCH):
        if c + NBUF - 1 < NCH:
            pending.append(start(c + NBUF - 1))
        ha, hb = pending.pop(0)
        ha.wait()
        hb.wait()
        slot = c % NBUF

        def group_body(g, carry, slot=slot, c=c):
            rvec = _dot_group(a_v, b_v, slot, g * LANES)
            out_v[pl.ds(c * CH + g * LANES, LANES)] = rvec
            return carry

        lax.fori_loop(0, CH // LANES, group_body, 0)

    pltpu.sync_copy(out_v, out_hbm.at[pl.ds(base, BPW)])


def kernel(center, context, W_in, W_out):
    return _w2v_kernel(center, context, W_in, W_out)


# speedup vs baseline: 1.0635x; 1.0635x over previous
import functools

import jax
import jax.numpy as jnp
from jax import lax
from jax.experimental import pallas as pl
from jax.experimental.pallas import tpu as pltpu
from jax.experimental.pallas import tpu_sc as plsc

VOCAB = 100000
DIM = 128
BATCH = 16384

NUM_CORES = 2
NUM_SUBCORES = 16
NW = NUM_CORES * NUM_SUBCORES
BPW = BATCH // NW
CH = 128
NCH = BPW // CH
NBUF = 2
LANES = 16


PSTRIDE = LANES + 1
PGROUP = PSTRIDE * LANES


def _lane_permute(x, idx):
    dnums = lax.GatherDimensionNumbers(
        offset_dims=(), collapsed_slice_dims=(0,), start_index_map=(0,))
    return lax.gather(
        x, idx[:, None], dnums, slice_sizes=(1,),
        mode=lax.GatherScatterMode.PROMISE_IN_BOUNDS)


def _dot_group(a_v, b_v, slot, row0):
    lanes = lax.broadcasted_iota(jnp.int32, (LANES,), 0)
    perms = [lanes ^ step for step in (8, 4, 2, 1)]

    @plsc.parallel_loop(0, LANES, unroll=2, carry=jnp.zeros((LANES,), jnp.float32))
    def res(r, out):
        row = row0 + r
        acc = None
        for k in range(DIM // LANES):
            av = a_v[slot, row, pl.ds(k * LANES, LANES)]
            bv = b_v[slot, row, pl.ds(k * LANES, LANES)]
            prod = av * bv
            acc = prod if acc is None else acc + prod
        for perm in perms:
            acc = acc + _lane_permute(acc, perm)
        return jnp.where(lanes == r, acc, out)

    return res


def _dot_group_cols(a_v, b_v, row0):
    rows = lax.broadcasted_iota(jnp.int32, (LANES,), 0) + row0
    zeros = jnp.zeros((LANES,), jnp.float32)
    nacc = 8

    def body(i, accs):
        d0 = i * nacc
        col = jnp.full((LANES,), d0, dtype=jnp.int32)
        outs = []
        for j, acc in enumerate(accs):
            cj = col + j
            av = plsc.load_gather(a_v, [rows, cj])
            bv = plsc.load_gather(b_v, [rows, cj])
            outs.append(acc + av * bv)
        return tuple(outs)

    accs = lax.fori_loop(0, DIM // nacc, body, (zeros,) * nacc)
    s01 = (accs[0] + accs[1]) + (accs[2] + accs[3])
    s23 = (accs[4] + accs[5]) + (accs[6] + accs[7])
    return s01 + s23


@functools.partial(
    pl.kernel,
    mesh=plsc.VectorSubcoreMesh(core_axis_name="c", subcore_axis_name="s"),
    out_type=jax.ShapeDtypeStruct((BATCH,), jnp.float32),
    scratch_types=[
        pltpu.VMEM((BPW,), jnp.int32),
        pltpu.VMEM((BPW,), jnp.int32),
        pltpu.VMEM((NBUF, CH, DIM), jnp.float32),
        pltpu.VMEM((NBUF, CH, DIM), jnp.float32),
        pltpu.VMEM((BPW,), jnp.float32),
        pltpu.SemaphoreType.DMA,
        pltpu.SemaphoreType.DMA,
    ],
    compiler_params=pltpu.CompilerParams(needs_layout_passes=False),
)
def _w2v_kernel(center_hbm, context_hbm, w_in_hbm, w_out_hbm, out_hbm,
                cidx_v, xidx_v, a_v, b_v, out_v, sem0, sem1):
    wid = lax.axis_index("s") * NUM_CORES + lax.axis_index("c")
    base = wid * BPW

    sems = (sem0, sem1)

    hc = pltpu.async_copy(center_hbm.at[pl.ds(base, BPW)], cidx_v, sem0)
    hx = pltpu.async_copy(context_hbm.at[pl.ds(base, BPW)], xidx_v, sem1)
    hc.wait()
    hx.wait()

    def start(c):
        slot = c % NBUF
        ha = pltpu.async_copy(
            w_in_hbm.at[cidx_v.at[pl.ds(c * CH, CH)]], a_v.at[slot], sems[slot])
        hb = pltpu.async_copy(
            w_out_hbm.at[xidx_v.at[pl.ds(c * CH, CH)]], b_v.at[slot], sems[slot])
        return ha, hb

    pending = [start(c) for c in range(NBUF - 1)]
    for c in range(NCH):
        if c + NBUF - 1 < NCH:
            pending.append(start(c + NBUF - 1))
        ha, hb = pending.pop(0)
        ha.wait()
        hb.wait()
        slot = c % NBUF

        def group_body(g, carry, slot=slot, c=c):
            rvec = _dot_group(a_v, b_v, slot, g * LANES)
            out_v[pl.ds(c * CH + g * LANES, LANES)] = rvec
            return carry

        lax.fori_loop(0, CH // LANES, group_body, 0)

    pltpu.sync_copy(out_v, out_hbm.at[pl.ds(base, BPW)])


def kernel(center, context, W_in, W_out):
    return _w2v_kernel(center, context, W_in, W_out)


# speedup vs baseline: 1.0740x; 1.0099x over previous
import functools

import jax
import jax.numpy as jnp
from jax import lax
from jax.experimental import pallas as pl
from jax.experimental.pallas import tpu as pltpu
from jax.experimental.pallas import tpu_sc as plsc

VOCAB = 100000
DIM = 128
BATCH = 16384

NUM_CORES = 2
NUM_SUBCORES = 16
NW = NUM_CORES * NUM_SUBCORES
BPW = BATCH // NW
CH = 128
NCH = BPW // CH
NBUF = 2
LANES = 16


PSTRIDE = LANES + 1
PGROUP = PSTRIDE * LANES


def _lane_permute(x, idx):
    dnums = lax.GatherDimensionNumbers(
        offset_dims=(), collapsed_slice_dims=(0,), start_index_map=(0,))
    return lax.gather(
        x, idx[:, None], dnums, slice_sizes=(1,),
        mode=lax.GatherScatterMode.PROMISE_IN_BOUNDS)


def _dot_group(a_v, b_v, slot, row0):
    lanes = lax.broadcasted_iota(jnp.int32, (LANES,), 0)
    perms = [lanes ^ step for step in (8, 4, 2, 1)]

    @plsc.parallel_loop(0, LANES, unroll=1, carry=jnp.zeros((LANES,), jnp.float32))
    def res(r, out):
        row = row0 + r
        acc = None
        for k in range(DIM // LANES):
            av = a_v[slot, row, pl.ds(k * LANES, LANES)]
            bv = b_v[slot, row, pl.ds(k * LANES, LANES)]
            prod = av * bv
            acc = prod if acc is None else acc + prod
        for perm in perms:
            acc = acc + _lane_permute(acc, perm)
        return jnp.where(lanes == r, acc, out)

    return res


def _dot_group_cols(a_v, b_v, row0):
    rows = lax.broadcasted_iota(jnp.int32, (LANES,), 0) + row0
    zeros = jnp.zeros((LANES,), jnp.float32)
    nacc = 8

    def body(i, accs):
        d0 = i * nacc
        col = jnp.full((LANES,), d0, dtype=jnp.int32)
        outs = []
        for j, acc in enumerate(accs):
            cj = col + j
            av = plsc.load_gather(a_v, [rows, cj])
            bv = plsc.load_gather(b_v, [rows, cj])
            outs.append(acc + av * bv)
        return tuple(outs)

    accs = lax.fori_loop(0, DIM // nacc, body, (zeros,) * nacc)
    s01 = (accs[0] + accs[1]) + (accs[2] + accs[3])
    s23 = (accs[4] + accs[5]) + (accs[6] + accs[7])
    return s01 + s23


@functools.partial(
    pl.kernel,
    mesh=plsc.VectorSubcoreMesh(core_axis_name="c", subcore_axis_name="s"),
    out_type=jax.ShapeDtypeStruct((BATCH,), jnp.float32),
    scratch_types=[
        pltpu.VMEM((BPW,), jnp.int32),
        pltpu.VMEM((BPW,), jnp.int32),
        pltpu.VMEM((NBUF, CH, DIM), jnp.float32),
        pltpu.VMEM((NBUF, CH, DIM), jnp.float32),
        pltpu.VMEM((BPW,), jnp.float32),
        pltpu.SemaphoreType.DMA,
        pltpu.SemaphoreType.DMA,
    ],
    compiler_params=pltpu.CompilerParams(needs_layout_passes=False),
)
def _w2v_kernel(center_hbm, context_hbm, w_in_hbm, w_out_hbm, out_hbm,
                cidx_v, xidx_v, a_v, b_v, out_v, sem0, sem1):
    wid = lax.axis_index("s") * NUM_CORES + lax.axis_index("c")
    base = wid * BPW

    sems = (sem0, sem1)

    hc = pltpu.async_copy(center_hbm.at[pl.ds(base, BPW)], cidx_v, sem0)
    hx = pltpu.async_copy(context_hbm.at[pl.ds(base, BPW)], xidx_v, sem1)
    hc.wait()
    hx.wait()

    def start(c):
        slot = c % NBUF
        ha = pltpu.async_copy(
            w_in_hbm.at[cidx_v.at[pl.ds(c * CH, CH)]], a_v.at[slot], sems[slot])
        hb = pltpu.async_copy(
            w_out_hbm.at[xidx_v.at[pl.ds(c * CH, CH)]], b_v.at[slot], sems[slot])
        return ha, hb

    pending = [start(c) for c in range(NBUF - 1)]
    for c in range(NCH):
        if c + NBUF - 1 < NCH:
            pending.append(start(c + NBUF - 1))
        ha, hb = pending.pop(0)
        ha.wait()
        hb.wait()
        slot = c % NBUF

        def group_body(g, carry, slot=slot, c=c):
            rvec = _dot_group(a_v, b_v, slot, g * LANES)
            out_v[pl.ds(c * CH + g * LANES, LANES)] = rvec
            return carry

        lax.fori_loop(0, CH // LANES, group_body, 0)

    pltpu.sync_copy(out_v, out_hbm.at[pl.ds(base, BPW)])


def kernel(center, context, W_in, W_out):
    return _w2v_kernel(center, context, W_in, W_out)


# speedup vs baseline: 1.0756x; 1.0015x over previous
import functools

import jax
import jax.numpy as jnp
from jax import lax
from jax.experimental import pallas as pl
from jax.experimental.pallas import tpu as pltpu
from jax.experimental.pallas import tpu_sc as plsc

VOCAB = 100000
DIM = 128
BATCH = 16384

NUM_CORES = 2
NUM_SUBCORES = 16
NW = NUM_CORES * NUM_SUBCORES
BPW = BATCH // NW
CH = 128
NCH = BPW // CH
NBUF = 2
LANES = 16


PSTRIDE = LANES + 1
PGROUP = PSTRIDE * LANES


def _lane_permute(x, idx):
    dnums = lax.GatherDimensionNumbers(
        offset_dims=(), collapsed_slice_dims=(0,), start_index_map=(0,))
    return lax.gather(
        x, idx[:, None], dnums, slice_sizes=(1,),
        mode=lax.GatherScatterMode.PROMISE_IN_BOUNDS)


def _dot_chunk(a_v, b_v, out_v, slot, out0):
    lanes = lax.broadcasted_iota(jnp.int32, (LANES,), 0)
    perms = [lanes ^ step for step in (8, 4, 2, 1)]
    lane0 = lanes == 0

    @plsc.parallel_loop(0, CH)
    def _(row):
        acc = None
        for k in range(DIM // LANES):
            av = a_v[slot, row, pl.ds(k * LANES, LANES)]
            bv = b_v[slot, row, pl.ds(k * LANES, LANES)]
            prod = av * bv
            acc = prod if acc is None else acc + prod
        for perm in perms:
            acc = acc + _lane_permute(acc, perm)
        plsc.store_scatter(out_v, [lanes + (out0 + row)], acc, mask=lane0)


def _dot_group_cols(a_v, b_v, row0):
    rows = lax.broadcasted_iota(jnp.int32, (LANES,), 0) + row0
    zeros = jnp.zeros((LANES,), jnp.float32)
    nacc = 8

    def body(i, accs):
        d0 = i * nacc
        col = jnp.full((LANES,), d0, dtype=jnp.int32)
        outs = []
        for j, acc in enumerate(accs):
            cj = col + j
            av = plsc.load_gather(a_v, [rows, cj])
            bv = plsc.load_gather(b_v, [rows, cj])
            outs.append(acc + av * bv)
        return tuple(outs)

    accs = lax.fori_loop(0, DIM // nacc, body, (zeros,) * nacc)
    s01 = (accs[0] + accs[1]) + (accs[2] + accs[3])
    s23 = (accs[4] + accs[5]) + (accs[6] + accs[7])
    return s01 + s23


@functools.partial(
    pl.kernel,
    mesh=plsc.VectorSubcoreMesh(core_axis_name="c", subcore_axis_name="s"),
    out_type=jax.ShapeDtypeStruct((BATCH,), jnp.float32),
    scratch_types=[
        pltpu.VMEM((BPW,), jnp.int32),
        pltpu.VMEM((BPW,), jnp.int32),
        pltpu.VMEM((NBUF, CH, DIM), jnp.float32),
        pltpu.VMEM((NBUF, CH, DIM), jnp.float32),
        pltpu.VMEM((BPW,), jnp.float32),
        pltpu.SemaphoreType.DMA,
        pltpu.SemaphoreType.DMA,
    ],
    compiler_params=pltpu.CompilerParams(needs_layout_passes=False),
)
def _w2v_kernel(center_hbm, context_hbm, w_in_hbm, w_out_hbm, out_hbm,
                cidx_v, xidx_v, a_v, b_v, out_v, sem0, sem1):
    wid = lax.axis_index("s") * NUM_CORES + lax.axis_index("c")
    base = wid * BPW

    sems = (sem0, sem1)

    hc = pltpu.async_copy(center_hbm.at[pl.ds(base, BPW)], cidx_v, sem0)
    hx = pltpu.async_copy(context_hbm.at[pl.ds(base, BPW)], xidx_v, sem1)
    hc.wait()
    hx.wait()

    def start(c):
        slot = c % NBUF
        ha = pltpu.async_copy(
            w_in_hbm.at[cidx_v.at[pl.ds(c * CH, CH)]], a_v.at[slot], sems[slot])
        hb = pltpu.async_copy(
            w_out_hbm.at[xidx_v.at[pl.ds(c * CH, CH)]], b_v.at[slot], sems[slot])
        return ha, hb

    pending = [start(c) for c in range(NBUF - 1)]
    for c in range(NCH):
        if c + NBUF - 1 < NCH:
            pending.append(start(c + NBUF - 1))
        ha, hb = pending.pop(0)
        ha.wait()
        hb.wait()
        slot = c % NBUF

        _dot_chunk(a_v, b_v, out_v, slot, c * CH)

    pltpu.sync_copy(out_v, out_hbm.at[pl.ds(base, BPW)])


def kernel(center, context, W_in, W_out):
    return _w2v_kernel(center, context, W_in, W_out)


# speedup vs baseline: 1.0844x; 1.0082x over previous
import functools

import jax
import jax.numpy as jnp
from jax import lax
from jax.experimental import pallas as pl
from jax.experimental.pallas import tpu as pltpu
from jax.experimental.pallas import tpu_sc as plsc

VOCAB = 100000
DIM = 128
BATCH = 16384

NUM_CORES = 2
NUM_SUBCORES = 16
NW = NUM_CORES * NUM_SUBCORES
BPW = BATCH // NW
CH = 128
NCH = BPW // CH
NBUF = 2
LANES = 16


PSTRIDE = LANES + 1
PGROUP = PSTRIDE * LANES


def _lane_permute(x, idx):
    dnums = lax.GatherDimensionNumbers(
        offset_dims=(), collapsed_slice_dims=(0,), start_index_map=(0,))
    return lax.gather(
        x, idx[:, None], dnums, slice_sizes=(1,),
        mode=lax.GatherScatterMode.PROMISE_IN_BOUNDS)


def _dot_chunk(a_v, b_v, out_v, slot, out0):
    lanes = lax.broadcasted_iota(jnp.int32, (LANES,), 0)
    perms = [lanes ^ step for step in (8, 4, 2, 1)]
    lane0 = lanes == 0

    @plsc.parallel_loop(0, CH)
    def _(row):
        acc = None
        for k in range(DIM // LANES):
            av = a_v[slot, row, pl.ds(k * LANES, LANES)]
            bv = b_v[slot, row, pl.ds(k * LANES, LANES)]
            prod = av * bv
            acc = prod if acc is None else acc + prod
        for perm in perms:
            acc = acc + _lane_permute(acc, perm)
        plsc.store_scatter(out_v, [lanes + (out0 + row)], acc, mask=lane0)


def _dot_group_cols(a_v, b_v, row0):
    rows = lax.broadcasted_iota(jnp.int32, (LANES,), 0) + row0
    zeros = jnp.zeros((LANES,), jnp.float32)
    nacc = 8

    def body(i, accs):
        d0 = i * nacc
        col = jnp.full((LANES,), d0, dtype=jnp.int32)
        outs = []
        for j, acc in enumerate(accs):
            cj = col + j
            av = plsc.load_gather(a_v, [rows, cj])
            bv = plsc.load_gather(b_v, [rows, cj])
            outs.append(acc + av * bv)
        return tuple(outs)

    accs = lax.fori_loop(0, DIM // nacc, body, (zeros,) * nacc)
    s01 = (accs[0] + accs[1]) + (accs[2] + accs[3])
    s23 = (accs[4] + accs[5]) + (accs[6] + accs[7])
    return s01 + s23


@functools.partial(
    pl.kernel,
    mesh=plsc.VectorSubcoreMesh(core_axis_name="c", subcore_axis_name="s"),
    out_type=jax.ShapeDtypeStruct((BATCH,), jnp.float32),
    scratch_types=[
        pltpu.VMEM((BPW,), jnp.int32),
        pltpu.VMEM((BPW,), jnp.int32),
        pltpu.VMEM((NBUF, CH, DIM), jnp.float32),
        pltpu.VMEM((NBUF, CH, DIM), jnp.float32),
        pltpu.VMEM((BPW,), jnp.float32),
        pltpu.SemaphoreType.DMA,
        pltpu.SemaphoreType.DMA,
        pltpu.SemaphoreType.DMA,
    ],
    compiler_params=pltpu.CompilerParams(needs_layout_passes=False),
)
def _w2v_kernel(center_hbm, context_hbm, w_in_hbm, w_out_hbm, out_hbm,
                cidx_v, xidx_v, a_v, b_v, out_v, sem0, sem1, sem2):
    wid = lax.axis_index("s") * NUM_CORES + lax.axis_index("c")
    base = wid * BPW

    sems = (sem0, sem1)

    hc = pltpu.async_copy(
        center_hbm.at[pl.ds(base, CH)], cidx_v.at[pl.ds(0, CH)], sem2)
    hx = pltpu.async_copy(
        context_hbm.at[pl.ds(base, CH)], xidx_v.at[pl.ds(0, CH)], sem2)
    hc.wait()
    hx.wait()

    def start(c):
        slot = c % NBUF
        ha = pltpu.async_copy(
            w_in_hbm.at[cidx_v.at[pl.ds(c * CH, CH)]], a_v.at[slot], sems[slot])
        hb = pltpu.async_copy(
            w_out_hbm.at[xidx_v.at[pl.ds(c * CH, CH)]], b_v.at[slot], sems[slot])
        return ha, hb

    pending = [start(0)]

    rest = BPW - CH
    hc = pltpu.async_copy(
        center_hbm.at[pl.ds(base + CH, rest)], cidx_v.at[pl.ds(CH, rest)], sem2)
    hx = pltpu.async_copy(
        context_hbm.at[pl.ds(base + CH, rest)], xidx_v.at[pl.ds(CH, rest)], sem2)
    hc.wait()
    hx.wait()

    hout = None
    for c in range(NCH):
        if c + NBUF - 1 < NCH:
            pending.append(start(c + NBUF - 1))
        ha, hb = pending.pop(0)
        ha.wait()
        hb.wait()
        slot = c % NBUF

        _dot_chunk(a_v, b_v, out_v, slot, c * CH)

        if hout is not None:
            hout.wait()
        hout = pltpu.async_copy(
            out_v.at[pl.ds(c * CH, CH)],
            out_hbm.at[pl.ds(base + c * CH, CH)], sem2)

    hout.wait()


def kernel(center, context, W_in, W_out):
    return _w2v_kernel(center, context, W_in, W_out)


# speedup vs baseline: 1.1002x; 1.0146x over previous
import functools

import jax
import jax.numpy as jnp
from jax import lax
from jax.experimental import pallas as pl
from jax.experimental.pallas import tpu as pltpu
from jax.experimental.pallas import tpu_sc as plsc

VOCAB = 100000
DIM = 128
BATCH = 16384

NUM_CORES = 2
NUM_SUBCORES = 16
NW = NUM_CORES * NUM_SUBCORES
BPW = BATCH // NW
CH = 128
NCH = BPW // CH
NBUF = 3
LANES = 16


PSTRIDE = LANES + 1
PGROUP = PSTRIDE * LANES


def _lane_permute(x, idx):
    dnums = lax.GatherDimensionNumbers(
        offset_dims=(), collapsed_slice_dims=(0,), start_index_map=(0,))
    return lax.gather(
        x, idx[:, None], dnums, slice_sizes=(1,),
        mode=lax.GatherScatterMode.PROMISE_IN_BOUNDS)


def _dot_chunk(a_v, b_v, out_v, slot, out0):
    lanes = lax.broadcasted_iota(jnp.int32, (LANES,), 0)
    perms = [lanes ^ step for step in (8, 4, 2, 1)]
    lane0 = lanes == 0

    @plsc.parallel_loop(0, CH)
    def _(row):
        acc = None
        for k in range(DIM // LANES):
            av = a_v[slot, row, pl.ds(k * LANES, LANES)]
            bv = b_v[slot, row, pl.ds(k * LANES, LANES)]
            prod = av * bv
            acc = prod if acc is None else acc + prod
        for perm in perms:
            acc = acc + _lane_permute(acc, perm)
        plsc.store_scatter(out_v, [lanes + (out0 + row)], acc, mask=lane0)


def _dot_group_cols(a_v, b_v, row0):
    rows = lax.broadcasted_iota(jnp.int32, (LANES,), 0) + row0
    zeros = jnp.zeros((LANES,), jnp.float32)
    nacc = 8

    def body(i, accs):
        d0 = i * nacc
        col = jnp.full((LANES,), d0, dtype=jnp.int32)
        outs = []
        for j, acc in enumerate(accs):
            cj = col + j
            av = plsc.load_gather(a_v, [rows, cj])
            bv = plsc.load_gather(b_v, [rows, cj])
            outs.append(acc + av * bv)
        return tuple(outs)

    accs = lax.fori_loop(0, DIM // nacc, body, (zeros,) * nacc)
    s01 = (accs[0] + accs[1]) + (accs[2] + accs[3])
    s23 = (accs[4] + accs[5]) + (accs[6] + accs[7])
    return s01 + s23


@functools.partial(
    pl.kernel,
    mesh=plsc.VectorSubcoreMesh(core_axis_name="c", subcore_axis_name="s"),
    out_type=jax.ShapeDtypeStruct((BATCH,), jnp.float32),
    scratch_types=[
        pltpu.VMEM((BPW,), jnp.int32),
        pltpu.VMEM((BPW,), jnp.int32),
        pltpu.VMEM((NBUF, CH, DIM), jnp.float32),
        pltpu.VMEM((NBUF, CH, DIM), jnp.float32),
        pltpu.VMEM((BPW,), jnp.float32),
        pltpu.SemaphoreType.DMA,
        pltpu.SemaphoreType.DMA,
        pltpu.SemaphoreType.DMA,
        pltpu.SemaphoreType.DMA,
    ],
    compiler_params=pltpu.CompilerParams(needs_layout_passes=False),
)
def _w2v_kernel(center_hbm, context_hbm, w_in_hbm, w_out_hbm, out_hbm,
                cidx_v, xidx_v, a_v, b_v, out_v, sem0, sem1, semx, sem2):
    wid = lax.axis_index("s") * NUM_CORES + lax.axis_index("c")
    base = wid * BPW

    sems = (sem0, sem1, semx)

    hc = pltpu.async_copy(
        center_hbm.at[pl.ds(base, CH)], cidx_v.at[pl.ds(0, CH)], sem2)
    hx = pltpu.async_copy(
        context_hbm.at[pl.ds(base, CH)], xidx_v.at[pl.ds(0, CH)], sem2)
    hc.wait()
    hx.wait()

    def start(c):
        slot = c % NBUF
        ha = pltpu.async_copy(
            w_in_hbm.at[cidx_v.at[pl.ds(c * CH, CH)]], a_v.at[slot], sems[slot])
        hb = pltpu.async_copy(
            w_out_hbm.at[xidx_v.at[pl.ds(c * CH, CH)]], b_v.at[slot], sems[slot])
        return ha, hb

    pending = [start(0)]

    rest = BPW - CH
    hc = pltpu.async_copy(
        center_hbm.at[pl.ds(base + CH, rest)], cidx_v.at[pl.ds(CH, rest)], sem2)
    hx = pltpu.async_copy(
        context_hbm.at[pl.ds(base + CH, rest)], xidx_v.at[pl.ds(CH, rest)], sem2)
    hc.wait()
    hx.wait()
    for j in range(1, NBUF - 1):
        pending.append(start(j))

    hout = None
    for c in range(NCH):
        if c + NBUF - 1 < NCH:
            pending.append(start(c + NBUF - 1))
        ha, hb = pending.pop(0)
        ha.wait()
        hb.wait()
        slot = c % NBUF

        _dot_chunk(a_v, b_v, out_v, slot, c * CH)

        if hout is not None:
            hout.wait()
        hout = pltpu.async_copy(
            out_v.at[pl.ds(c * CH, CH)],
            out_hbm.at[pl.ds(base + c * CH, CH)], sem2)

    hout.wait()


def kernel(center, context, W_in, W_out):
    return _w2v_kernel(center, context, W_in, W_out)
